# baseline (speedup 1.0000x reference)
import functools

import jax
import jax.numpy as jnp
from jax import lax
from jax.experimental import pallas as pl
from jax.experimental.pallas import tpu as pltpu
from jax.experimental.pallas import tpu_sc as plsc

_N = 10000
_E = 320000
_D = 128
_NPAD = 10240
_NG = _NPAD // 128
_NW = 32
_CHUNK = 96
_WSZ = 15
_CPT = 105
_NWIN = _CPT // _WSZ
_EPT = _CPT * _CHUNK
_EP = _NW * _EPT
_RPS = _NPAD // 16
_EPS = 1e-5
_F32 = jnp.float32

_mesh = plsc.VectorSubcoreMesh(core_axis_name="c", subcore_axis_name="s")
_HIGH = lax.Precision.HIGHEST


def _dot(a, b):
    return jnp.dot(a, b, precision=_HIGH, preferred_element_type=_F32)


@functools.partial(
    pl.kernel,
    out_type=jax.ShapeDtypeStruct((_NW, _NPAD), _F32),
    mesh=_mesh,
    compiler_params=pltpu.CompilerParams(needs_layout_passes=False),
    scratch_types=[
        pltpu.VMEM((_NWIN, 16, _CHUNK), jnp.int32),
        pltpu.VMEM((_NPAD,), _F32),
    ],
)
def _deg_kernel(src_hbm, out_hbm, src_v, degloc):
    c = lax.axis_index("c")
    s = lax.axis_index("s")
    wid = c * 16 + s
    pltpu.sync_copy(src_hbm.at[wid], src_v)
    zeros16 = jnp.zeros((16,), _F32)
    ones16 = jnp.ones((16,), _F32)

    def zbody(i, _):
        degloc[pl.ds(i * 16, 16)] = zeros16
        return ()

    lax.fori_loop(0, _NPAD // 16, zbody, ())

    def ebody(ci, _):
        w = ci // _WSZ
        r = ci % _WSZ
        for v in range(_CHUNK // 16):
            idx = src_v[w, r, pl.ds(v * 16, 16)]
            plsc.addupdate_scatter(degloc, [idx], ones16)
        return ()

    lax.fori_loop(0, _CPT, ebody, ())
    pltpu.sync_copy(degloc, out_hbm.at[wid])


@functools.partial(
    pl.kernel,
    out_type=jax.ShapeDtypeStruct((2, _NPAD, _D), _F32),
    mesh=_mesh,
    scratch_types=[
        pltpu.VMEM((2, 16, _CHUNK), jnp.int32),
        pltpu.VMEM((2, 16, _CHUNK), jnp.int32),
        pltpu.VMEM((3, _CHUNK, _D), _F32),
        pltpu.VMEM_SHARED((_NPAD, _D), _F32),
        pltpu.SemaphoreType.DMA,
        pltpu.SemaphoreType.DMA,
        pltpu.SemaphoreType.DMA,
        pltpu.SemaphoreType.DMA,
        pltpu.SemaphoreType.DMA,
        pltpu.SemaphoreType.DMA,
        pltpu.SemaphoreType.DMA,
        pltpu.SemaphoreType.DMA,
    ],
)
def _scatter2_kernel(u_hbm, src_hbm, dst_hbm, zeros_hbm, out_hbm,
                     src_w, dst_w, rows_v, acc,
                     g0, g1, g2, t0, t1, t2, iw, zsem):
    gsem = (g0, g1, g2)
    ssem = (t0, t1, t2)
    c = lax.axis_index("c")
    s = lax.axis_index("s")
    wid = c * 16 + s

    pltpu.async_copy(zeros_hbm.at[pl.ds(s * _RPS, _RPS)],
                     acc.at[pl.ds(s * _RPS, _RPS)], zsem)

    def start_w(w, slot):
        pltpu.async_copy(src_hbm.at[wid, w], src_w.at[slot], iw)
        pltpu.async_copy(dst_hbm.at[wid, w], dst_w.at[slot], iw)

    def wait_w(w, slot):
        pltpu.make_async_copy(src_hbm.at[wid, w], src_w.at[slot],
                              iw).wait()
        pltpu.make_async_copy(dst_hbm.at[wid, w], dst_w.at[slot],
                              iw).wait()

    def gidx(ref, w, r):
        return ref.at[w & 1, r]

    def start_g(w, r, k):
        pltpu.async_copy(u_hbm.at[gidx(src_w, w, r)], rows_v.at[k],
                         gsem[k])

    def wait_g(w, r, k):
        pltpu.make_async_copy(u_hbm.at[gidx(src_w, w, r)], rows_v.at[k],
                              gsem[k]).wait()

    def start_s(w, r, k):
        pltpu.async_copy(rows_v.at[k], acc.at[gidx(dst_w, w, r)],
                         ssem[k], add=True)

    def wait_s(w, r, k):
        pltpu.make_async_copy(rows_v.at[k], acc.at[gidx(dst_w, w, r)],
                              ssem[k]).wait()

    pltpu.sync_copy(src_hbm.at[wid, 0], src_w.at[0])
    pltpu.sync_copy(dst_hbm.at[wid, 0], dst_w.at[0])
    start_w(1, 1)
    pltpu.make_async_copy(zeros_hbm.at[pl.ds(s * _RPS, _RPS)],
                          acc.at[pl.ds(s * _RPS, _RPS)], zsem).wait()
    plsc.subcore_barrier()

    start_g(0, 0, 0)
    start_g(0, 1, 1)

    def wbody(w, _):
        for r in range(_WSZ):
            wait_g(w, r, r % 3)
            if r < _WSZ - 2:
                start_g(w, r + 2, (r + 2) % 3)
            elif r == _WSZ - 2:
                @pl.when(w + 1 < _NWIN)
                def _():
                    wait_w(w + 1, (w + 1) & 1)
                    start_g(w + 1, 0, 0)
            else:
                @pl.when(w + 1 < _NWIN)
                def _():
                    start_g(w + 1, 1, 1)
            pltpu.sync_copy(rows_v.at[r % 3],
                            acc.at[gidx(dst_w, w, r)], add=True)

        @pl.when(w + 2 < _NWIN)
        def _():
            start_w(w + 2, w & 1)
        return ()

    lax.fori_loop(0, _NWIN, wbody, ())
    plsc.subcore_barrier()
    pltpu.sync_copy(acc.at[pl.ds(s * _RPS, _RPS)],
                    out_hbm.at[c, pl.ds(s * _RPS, _RPS)])


def _dinv_body(degp_ref, dinv_ref, nd2_ref):
    deg = jnp.sum(degp_ref[...], axis=0)
    dinv = jnp.where(deg > 0.0, lax.rsqrt(deg), 0.0)
    dinv_ref[...] = dinv
    nd2_ref[...] = -(dinv * dinv)


_dinv_call = pl.pallas_call(
    _dinv_body,
    out_shape=[jax.ShapeDtypeStruct((_NG, _D), _F32),
               jax.ShapeDtypeStruct((_NG, _D), _F32)],
)


def _scale_body(x_ref, s_ref, o_ref):
    o_ref[...] = x_ref[...] * s_ref[...]


_scale_call = pl.pallas_call(
    _scale_body,
    grid=(8,),
    in_specs=[
        pl.BlockSpec((_NPAD // 8, _D), lambda i: (i, 0)),
        pl.BlockSpec((_NPAD // 8, 1), lambda i: (i, 0)),
    ],
    out_specs=pl.BlockSpec((_NPAD // 8, _D), lambda i: (i, 0)),
    out_shape=jax.ShapeDtypeStruct((_NPAD, _D), _F32),
)


def _mid_body(p_ref, dinv_ref, nd2_ref, s1_ref, u2_ref):
    P = p_ref[0] + p_ref[1]
    s1 = P * dinv_ref[...]
    s1_ref[...] = s1
    u2_ref[...] = P * nd2_ref[...]


_mid_call = pl.pallas_call(
    _mid_body,
    grid=(8,),
    in_specs=[
        pl.BlockSpec((2, _NPAD // 8, _D), lambda i: (0, i, 0)),
        pl.BlockSpec((_NPAD // 8, 1), lambda i: (i, 0)),
        pl.BlockSpec((_NPAD // 8, 1), lambda i: (i, 0)),
    ],
    out_specs=[
        pl.BlockSpec((_NPAD // 8, _D), lambda i: (i, 0)),
        pl.BlockSpec((_NPAD // 8, _D), lambda i: (i, 0)),
    ],
    out_shape=[jax.ShapeDtypeStruct((_NPAD, _D), _F32),
               jax.ShapeDtypeStruct((_NPAD, _D), _F32)],
)


_CB = 2048
_NB = _NPAD // _CB


def _cheb_phase(i, h_ref, s1_ref, p2_ref, dinv_ref, W_ref, b_ref,
                outbuf, ssum, sqsum, hbuf=None):
    h = h_ref[...]
    s2 = (p2_ref[0] + p2_ref[1]) * dinv_ref[...]
    out = (_dot(h, W_ref[0] - W_ref[2]) - _dot(s1_ref[...], W_ref[1])
           - 2.0 * _dot(s2, W_ref[2]) + b_ref[...])
    outbuf[pl.ds(i * _CB, _CB), :] = out
    if hbuf is not None:
        hbuf[pl.ds(i * _CB, _CB), :] = h
    mask = (i * _CB + lax.broadcasted_iota(jnp.int32, (_CB, 1), 0)) < _N
    om = jnp.where(mask, out, 0.0)

    @pl.when(i == 0)
    def _():
        ssum[...] = jnp.zeros((1, _D), _F32)
        sqsum[...] = jnp.zeros((1, _D), _F32)

    ssum[...] += jnp.sum(om, axis=0, keepdims=True)
    sqsum[...] += jnp.sum(om * om, axis=0, keepdims=True)


def _bn_relu(out, ssum, sqsum, g, be):
    mean = ssum / _N
    var = sqsum / _N - mean * mean
    outn = (out - mean) * lax.rsqrt(var + _EPS) * g + be
    return jnp.maximum(outn, 0.0)


def _layer2_body(h_ref, s1_ref, p2_ref, dinv_ref, W_ref, b_ref, MW_ref,
                 Mb_ref, g_ref, be_ref, hn_ref, u1_ref,
                 outbuf, hbuf, ssum, sqsum):
    p = pl.program_id(0)
    i = pl.program_id(1)

    @pl.when(p == 0)
    def _():
        _cheb_phase(i, h_ref, s1_ref, p2_ref, dinv_ref, W_ref, b_ref,
                    outbuf, ssum, sqsum, hbuf=hbuf)

    @pl.when(p == 1)
    def _():
        outn = _bn_relu(outbuf[pl.ds(i * _CB, _CB), :], ssum[...],
                        sqsum[...], g_ref[...], be_ref[...])
        hn = (outn + _dot(hbuf[pl.ds(i * _CB, _CB), :], MW_ref[...])
              + Mb_ref[...])
        hn_ref[...] = hn
        u1_ref[...] = hn * dinv_ref[...]


def _park(p, i):
    return (1 - p) * i + p * (_NB - 1)


_layer2_call = pl.pallas_call(
    _layer2_body,
    grid=(2, _NB),
    in_specs=[
        pl.BlockSpec((_CB, _D), lambda p, i: (_park(p, i), 0)),
        pl.BlockSpec((_CB, _D), lambda p, i: (_park(p, i), 0)),
        pl.BlockSpec((2, _CB, _D), lambda p, i: (0, _park(p, i), 0)),
        pl.BlockSpec((_CB, 1), lambda p, i: (i, 0)),
        pl.BlockSpec((3, _D, _D), lambda p, i: (0, 0, 0)),
        pl.BlockSpec((1, _D), lambda p, i: (0, 0)),
        pl.BlockSpec((_D, _D), lambda p, i: (0, 0)),
        pl.BlockSpec((1, _D), lambda p, i: (0, 0)),
        pl.BlockSpec((1, _D), lambda p, i: (0, 0)),
        pl.BlockSpec((1, _D), lambda p, i: (0, 0)),
    ],
    out_specs=[
        pl.BlockSpec((_CB, _D), lambda p, i: (p * i, 0)),
        pl.BlockSpec((_CB, _D), lambda p, i: (p * i, 0)),
    ],
    out_shape=[jax.ShapeDtypeStruct((_NPAD, _D), _F32),
               jax.ShapeDtypeStruct((_NPAD, _D), _F32)],
    scratch_shapes=[
        pltpu.VMEM((_NPAD, _D), _F32),
        pltpu.VMEM((_NPAD, _D), _F32),
        pltpu.VMEM((1, _D), _F32),
        pltpu.VMEM((1, _D), _F32),
    ],
)


def _final2_body(h_ref, s1_ref, p2_ref, dinv_ref, W_ref, b_ref, g_ref,
                 be_ref, o_ref, outbuf, ssum, sqsum):
    p = pl.program_id(0)
    i = pl.program_id(1)

    @pl.when(p == 0)
    def _():
        _cheb_phase(i, h_ref, s1_ref, p2_ref, dinv_ref, W_ref, b_ref,
                    outbuf, ssum, sqsum)

    @pl.when(p == 1)
    def _():
        u = _bn_relu(outbuf[pl.ds(i * _CB, _CB), :], ssum[...],
                     sqsum[...], g_ref[...], be_ref[...])
        u = u - jnp.max(u, axis=1, keepdims=True)
        o_ref[...] = u - jnp.log(jnp.sum(jnp.exp(u), axis=1,
                                         keepdims=True))


_final2_call = pl.pallas_call(
    _final2_body,
    grid=(2, _NB),
    in_specs=[
        pl.BlockSpec((_CB, _D), lambda p, i: (_park(p, i), 0)),
        pl.BlockSpec((_CB, _D), lambda p, i: (_park(p, i), 0)),
        pl.BlockSpec((2, _CB, _D), lambda p, i: (0, _park(p, i), 0)),
        pl.BlockSpec((_CB, 1), lambda p, i: (_park(p, i), 0)),
        pl.BlockSpec((3, _D, _D), lambda p, i: (0, 0, 0)),
        pl.BlockSpec((1, _D), lambda p, i: (0, 0)),
        pl.BlockSpec((1, _D), lambda p, i: (0, 0)),
        pl.BlockSpec((1, _D), lambda p, i: (0, 0)),
    ],
    out_specs=pl.BlockSpec((_CB, _D), lambda p, i: (p * i, 0)),
    out_shape=jax.ShapeDtypeStruct((_NPAD, _D), _F32),
    scratch_shapes=[
        pltpu.VMEM((_NPAD, _D), _F32),
        pltpu.VMEM((1, _D), _F32),
        pltpu.VMEM((1, _D), _F32),
    ],
)


def kernel(edge_index, x, W0, b0, g0, be0, MW0, Mb0, W1, b1, g1, be1,
           MW1, Mb1, W2, b2, g2, be2, MW2, Mb2, W3, b3, g3, be3):
    src = edge_index[0]
    dst = edge_index[1]
    pad = _N + (jnp.arange(_EP - _E, dtype=jnp.int32) % (_NPAD - _N))
    fill = jnp.zeros((_NW, _NWIN, 1, _CHUNK), jnp.int32)
    src_p = jnp.concatenate([src, pad]).reshape(_NW, _NWIN, _WSZ, _CHUNK)
    src_p = jnp.concatenate([src_p, fill], axis=2)
    dst_p = jnp.concatenate([dst, pad]).reshape(_NW, _NWIN, _WSZ, _CHUNK)
    dst_p = jnp.concatenate([dst_p, fill], axis=2)
    x_p = jnp.concatenate(
        [x, jnp.zeros((_NPAD - _N, _D), x.dtype)], axis=0)
    zrows = jnp.zeros((_NPAD, _D), _F32)

    degp = _deg_kernel(src_p).reshape(_NW, _NG, _D)
    dinv80, nd280 = _dinv_call(degp)
    dinvcol = dinv80.reshape(_NPAD, 1)
    nd2col = nd280.reshape(_NPAD, 1)

    row = lambda v: v.reshape(1, _D)
    params = [
        (W0, row(b0), row(g0), row(be0), MW0, row(Mb0)),
        (W1, row(b1), row(g1), row(be1), MW1, row(Mb1)),
        (W2, row(b2), row(g2), row(be2), MW2, row(Mb2)),
        (W3, row(b3), row(g3), row(be3), None, None),
    ]

    h = x_p
    u = _scale_call(x_p, dinvcol)
    for l in range(4):
        W, b, g, be, MW, Mb = params[l]
        P1 = _scatter2_kernel(u, src_p, dst_p, zrows)
        s1, u2 = _mid_call(P1, dinvcol, nd2col)
        P2 = _scatter2_kernel(u2, src_p, dst_p, zrows)
        if l < 3:
            h, u = _layer2_call(h, s1, P2, dinvcol, W, b, MW, Mb, g, be)
        else:
            return _final2_call(h, s1, P2, dinvcol, W, b, g, be)[0:_N]


# speedup vs baseline: 1.0049x; 1.0049x over previous
import functools

import jax
import jax.numpy as jnp
from jax import lax
from jax.experimental import pallas as pl
from jax.experimental.pallas import tpu as pltpu
from jax.experimental.pallas import tpu_sc as plsc

_N = 10000
_E = 320000
_D = 128
_NPAD = 10240
_NG = _NPAD // 128
_NW = 32
_CHUNK = 96
_WSZ = 15
_CPT = 105
_NWIN = _CPT // _WSZ
_EPT = _CPT * _CHUNK
_EP = _NW * _EPT
_RPS = _NPAD // 16
_EPS = 1e-5
_F32 = jnp.float32

_mesh = plsc.VectorSubcoreMesh(core_axis_name="c", subcore_axis_name="s")
_HIGH = lax.Precision.HIGHEST


def _dot(a, b):
    return jnp.dot(a, b, precision=_HIGH, preferred_element_type=_F32)


@functools.partial(
    pl.kernel,
    out_type=jax.ShapeDtypeStruct((_NW, _NPAD), _F32),
    mesh=_mesh,
    compiler_params=pltpu.CompilerParams(needs_layout_passes=False),
    scratch_types=[
        pltpu.VMEM((_NWIN, 16, _CHUNK), jnp.int32),
        pltpu.VMEM((_NPAD,), _F32),
    ],
)
def _deg_kernel(src_hbm, out_hbm, src_v, degloc):
    c = lax.axis_index("c")
    s = lax.axis_index("s")
    wid = c * 16 + s
    pltpu.sync_copy(src_hbm.at[wid], src_v)
    zeros16 = jnp.zeros((16,), _F32)
    ones16 = jnp.ones((16,), _F32)

    def zbody(i, _):
        degloc[pl.ds(i * 16, 16)] = zeros16
        return ()

    lax.fori_loop(0, _NPAD // 16, zbody, ())

    def ebody(ci, _):
        w = ci // _WSZ
        r = ci % _WSZ
        for v in range(_CHUNK // 16):
            idx = src_v[w, r, pl.ds(v * 16, 16)]
            plsc.addupdate_scatter(degloc, [idx], ones16)
        return ()

    lax.fori_loop(0, _CPT, ebody, ())
    pltpu.sync_copy(degloc, out_hbm.at[wid])


@functools.partial(
    pl.kernel,
    out_type=jax.ShapeDtypeStruct((2, _NPAD, _D), _F32),
    mesh=_mesh,
    scratch_types=[
        pltpu.VMEM((2, 16, _CHUNK), jnp.int32),
        pltpu.VMEM((2, 16, _CHUNK), jnp.int32),
        pltpu.VMEM((3, _CHUNK, _D), _F32),
        pltpu.VMEM_SHARED((_NPAD, _D), _F32),
        pltpu.SemaphoreType.DMA,
        pltpu.SemaphoreType.DMA,
        pltpu.SemaphoreType.DMA,
        pltpu.SemaphoreType.DMA,
        pltpu.SemaphoreType.DMA,
    ],
)
def _scatter2_kernel(u_hbm, src_hbm, dst_hbm, zeros_hbm, out_hbm,
                     src_w, dst_w, rows_v, acc,
                     g0, g1, g2, iw, zsem):
    gsem = (g0, g1, g2)
    c = lax.axis_index("c")
    s = lax.axis_index("s")
    wid = c * 16 + s

    pltpu.async_copy(zeros_hbm.at[pl.ds(s * _RPS, _RPS)],
                     acc.at[pl.ds(s * _RPS, _RPS)], zsem)

    def start_w(w, slot):
        pltpu.async_copy(src_hbm.at[wid, w], src_w.at[slot], iw)
        pltpu.async_copy(dst_hbm.at[wid, w], dst_w.at[slot], iw)

    def wait_w(w, slot):
        pltpu.make_async_copy(src_hbm.at[wid, w], src_w.at[slot],
                              iw).wait()
        pltpu.make_async_copy(dst_hbm.at[wid, w], dst_w.at[slot],
                              iw).wait()

    def gidx(ref, w, r):
        return ref.at[w & 1, r]

    def start_g(w, r, k):
        pltpu.async_copy(u_hbm.at[gidx(src_w, w, r)], rows_v.at[k],
                         gsem[k])

    def wait_g(w, r, k):
        pltpu.make_async_copy(u_hbm.at[gidx(src_w, w, r)], rows_v.at[k],
                              gsem[k]).wait()

    pltpu.sync_copy(src_hbm.at[wid, 0], src_w.at[0])
    pltpu.sync_copy(dst_hbm.at[wid, 0], dst_w.at[0])
    start_w(1, 1)
    pltpu.make_async_copy(zeros_hbm.at[pl.ds(s * _RPS, _RPS)],
                          acc.at[pl.ds(s * _RPS, _RPS)], zsem).wait()
    plsc.subcore_barrier()

    start_g(0, 0, 0)
    start_g(0, 1, 1)

    def wbody(w, _):
        for r in range(_WSZ):
            wait_g(w, r, r % 3)
            if r < _WSZ - 2:
                start_g(w, r + 2, (r + 2) % 3)
            elif r == _WSZ - 2:
                @pl.when(w + 1 < _NWIN)
                def _():
                    wait_w(w + 1, (w + 1) & 1)
                    start_g(w + 1, 0, 0)
            else:
                @pl.when(w + 1 < _NWIN)
                def _():
                    start_g(w + 1, 1, 1)
            pltpu.sync_copy(rows_v.at[r % 3],
                            acc.at[gidx(dst_w, w, r)], add=True)

        @pl.when(w + 2 < _NWIN)
        def _():
            start_w(w + 2, w & 1)
        return ()

    lax.fori_loop(0, _NWIN, wbody, ())
    plsc.subcore_barrier()
    pltpu.sync_copy(acc.at[pl.ds(s * _RPS, _RPS)],
                    out_hbm.at[c, pl.ds(s * _RPS, _RPS)])


def _dinv_body(degp_ref, dinv_ref, nd2_ref):
    deg = jnp.sum(degp_ref[...], axis=0)
    dinv = jnp.where(deg > 0.0, lax.rsqrt(deg), 0.0)
    dinv_ref[...] = dinv
    nd2_ref[...] = -(dinv * dinv)


_dinv_call = pl.pallas_call(
    _dinv_body,
    out_shape=[jax.ShapeDtypeStruct((_NG, _D), _F32),
               jax.ShapeDtypeStruct((_NG, _D), _F32)],
)


def _scale_body(x_ref, s_ref, o_ref):
    o_ref[...] = x_ref[...] * s_ref[...]


_scale_call = pl.pallas_call(
    _scale_body,
    out_shape=jax.ShapeDtypeStruct((_NPAD, _D), _F32),
)


def _mid_body(p_ref, dinv_ref, nd2_ref, s1_ref, u2_ref):
    P = p_ref[0] + p_ref[1]
    s1 = P * dinv_ref[...]
    s1_ref[...] = s1
    u2_ref[...] = P * nd2_ref[...]


_mid_call = pl.pallas_call(
    _mid_body,
    out_shape=[jax.ShapeDtypeStruct((_NPAD, _D), _F32),
               jax.ShapeDtypeStruct((_NPAD, _D), _F32)],
)


_CB = 2048
_NB = _NPAD // _CB


def _cheb_phase(i, h_ref, s1_ref, p2_ref, dinv_ref, W_ref, b_ref,
                outbuf, ssum, sqsum, hbuf=None):
    h = h_ref[...]
    s2 = (p2_ref[0] + p2_ref[1]) * dinv_ref[...]
    out = (_dot(h, W_ref[0] - W_ref[2]) - _dot(s1_ref[...], W_ref[1])
           - 2.0 * _dot(s2, W_ref[2]) + b_ref[...])
    outbuf[pl.ds(i * _CB, _CB), :] = out
    if hbuf is not None:
        hbuf[pl.ds(i * _CB, _CB), :] = h
    mask = (i * _CB + lax.broadcasted_iota(jnp.int32, (_CB, 1), 0)) < _N
    om = jnp.where(mask, out, 0.0)

    @pl.when(i == 0)
    def _():
        ssum[...] = jnp.zeros((1, _D), _F32)
        sqsum[...] = jnp.zeros((1, _D), _F32)

    ssum[...] += jnp.sum(om, axis=0, keepdims=True)
    sqsum[...] += jnp.sum(om * om, axis=0, keepdims=True)


def _bn_relu(out, ssum, sqsum, g, be):
    mean = ssum / _N
    var = sqsum / _N - mean * mean
    outn = (out - mean) * lax.rsqrt(var + _EPS) * g + be
    return jnp.maximum(outn, 0.0)


def _layer2_body(h_ref, s1_ref, p2_ref, dinv_ref, W_ref, b_ref, MW_ref,
                 Mb_ref, g_ref, be_ref, hn_ref, u1_ref,
                 outbuf, hbuf, ssum, sqsum):
    p = pl.program_id(0)
    i = pl.program_id(1)

    @pl.when(p == 0)
    def _():
        _cheb_phase(i, h_ref, s1_ref, p2_ref, dinv_ref, W_ref, b_ref,
                    outbuf, ssum, sqsum, hbuf=hbuf)

    @pl.when(p == 1)
    def _():
        outn = _bn_relu(outbuf[pl.ds(i * _CB, _CB), :], ssum[...],
                        sqsum[...], g_ref[...], be_ref[...])
        hn = (outn + _dot(hbuf[pl.ds(i * _CB, _CB), :], MW_ref[...])
              + Mb_ref[...])
        hn_ref[...] = hn
        u1_ref[...] = hn * dinv_ref[...]


def _park(p, i):
    return (1 - p) * i + p * (_NB - 1)


_layer2_call = pl.pallas_call(
    _layer2_body,
    grid=(2, _NB),
    in_specs=[
        pl.BlockSpec((_CB, _D), lambda p, i: (_park(p, i), 0)),
        pl.BlockSpec((_CB, _D), lambda p, i: (_park(p, i), 0)),
        pl.BlockSpec((2, _CB, _D), lambda p, i: (0, _park(p, i), 0)),
        pl.BlockSpec((_CB, 1), lambda p, i: (i, 0)),
        pl.BlockSpec((3, _D, _D), lambda p, i: (0, 0, 0)),
        pl.BlockSpec((1, _D), lambda p, i: (0, 0)),
        pl.BlockSpec((_D, _D), lambda p, i: (0, 0)),
        pl.BlockSpec((1, _D), lambda p, i: (0, 0)),
        pl.BlockSpec((1, _D), lambda p, i: (0, 0)),
        pl.BlockSpec((1, _D), lambda p, i: (0, 0)),
    ],
    out_specs=[
        pl.BlockSpec((_CB, _D), lambda p, i: (p * i, 0)),
        pl.BlockSpec((_CB, _D), lambda p, i: (p * i, 0)),
    ],
    out_shape=[jax.ShapeDtypeStruct((_NPAD, _D), _F32),
               jax.ShapeDtypeStruct((_NPAD, _D), _F32)],
    scratch_shapes=[
        pltpu.VMEM((_NPAD, _D), _F32),
        pltpu.VMEM((_NPAD, _D), _F32),
        pltpu.VMEM((1, _D), _F32),
        pltpu.VMEM((1, _D), _F32),
    ],
)


def _final2_body(h_ref, s1_ref, p2_ref, dinv_ref, W_ref, b_ref, g_ref,
                 be_ref, o_ref, outbuf, ssum, sqsum):
    p = pl.program_id(0)
    i = pl.program_id(1)

    @pl.when(p == 0)
    def _():
        _cheb_phase(i, h_ref, s1_ref, p2_ref, dinv_ref, W_ref, b_ref,
                    outbuf, ssum, sqsum)

    @pl.when(p == 1)
    def _():
        u = _bn_relu(outbuf[pl.ds(i * _CB, _CB), :], ssum[...],
                     sqsum[...], g_ref[...], be_ref[...])
        u = u - jnp.max(u, axis=1, keepdims=True)
        o_ref[...] = u - jnp.log(jnp.sum(jnp.exp(u), axis=1,
                                         keepdims=True))


_final2_call = pl.pallas_call(
    _final2_body,
    grid=(2, _NB),
    in_specs=[
        pl.BlockSpec((_CB, _D), lambda p, i: (_park(p, i), 0)),
        pl.BlockSpec((_CB, _D), lambda p, i: (_park(p, i), 0)),
        pl.BlockSpec((2, _CB, _D), lambda p, i: (0, _park(p, i), 0)),
        pl.BlockSpec((_CB, 1), lambda p, i: (_park(p, i), 0)),
        pl.BlockSpec((3, _D, _D), lambda p, i: (0, 0, 0)),
        pl.BlockSpec((1, _D), lambda p, i: (0, 0)),
        pl.BlockSpec((1, _D), lambda p, i: (0, 0)),
        pl.BlockSpec((1, _D), lambda p, i: (0, 0)),
    ],
    out_specs=pl.BlockSpec((_CB, _D), lambda p, i: (p * i, 0)),
    out_shape=jax.ShapeDtypeStruct((_NPAD, _D), _F32),
    scratch_shapes=[
        pltpu.VMEM((_NPAD, _D), _F32),
        pltpu.VMEM((1, _D), _F32),
        pltpu.VMEM((1, _D), _F32),
    ],
)


def kernel(edge_index, x, W0, b0, g0, be0, MW0, Mb0, W1, b1, g1, be1,
           MW1, Mb1, W2, b2, g2, be2, MW2, Mb2, W3, b3, g3, be3):
    src = edge_index[0]
    dst = edge_index[1]
    pad = _N + (jnp.arange(_EP - _E, dtype=jnp.int32) % (_NPAD - _N))
    fill = jnp.zeros((_NW, _NWIN, 1, _CHUNK), jnp.int32)
    src_p = jnp.concatenate([src, pad]).reshape(_NW, _NWIN, _WSZ, _CHUNK)
    src_p = jnp.concatenate([src_p, fill], axis=2)
    dst_p = jnp.concatenate([dst, pad]).reshape(_NW, _NWIN, _WSZ, _CHUNK)
    dst_p = jnp.concatenate([dst_p, fill], axis=2)
    x_p = jnp.concatenate(
        [x, jnp.zeros((_NPAD - _N, _D), x.dtype)], axis=0)
    zrows = jnp.zeros((_NPAD, _D), _F32)

    degp = _deg_kernel(src_p).reshape(_NW, _NG, _D)
    dinv80, nd280 = _dinv_call(degp)
    dinvcol = dinv80.reshape(_NPAD, 1)
    nd2col = nd280.reshape(_NPAD, 1)

    row = lambda v: v.reshape(1, _D)
    params = [
        (W0, row(b0), row(g0), row(be0), MW0, row(Mb0)),
        (W1, row(b1), row(g1), row(be1), MW1, row(Mb1)),
        (W2, row(b2), row(g2), row(be2), MW2, row(Mb2)),
        (W3, row(b3), row(g3), row(be3), None, None),
    ]

    h = x_p
    u = _scale_call(x_p, dinvcol)
    for l in range(4):
        W, b, g, be, MW, Mb = params[l]
        P1 = _scatter2_kernel(u, src_p, dst_p, zrows)
        s1, u2 = _mid_call(P1, dinvcol, nd2col)
        P2 = _scatter2_kernel(u2, src_p, dst_p, zrows)
        if l < 3:
            h, u = _layer2_call(h, s1, P2, dinvcol, W, b, MW, Mb, g, be)
        else:
            return _final2_call(h, s1, P2, dinvcol, W, b, g, be)[0:_N]
